# baseline (device time: 249212 ns/iter reference)
import jax
import jax.numpy as jnp
from jax import lax
from jax.experimental import pallas as pl
from jax.experimental.pallas import tpu as pltpu

N_DEV = 32
M_PER = 128


def kernel(x, w_mat, scale_x, scale_w):
    m_per, k = x.shape
    _, n_per = w_mat.shape

    def body(x_ref, w_ref, sx_ref, sw_ref, out_ref, xf_ref, send_sems, recv_sems):
        my = lax.axis_index("i")
        left = lax.rem(my + N_DEV - 1, N_DEV)
        right = lax.rem(my + 1, N_DEV)

        barrier_sem = pltpu.get_barrier_semaphore()
        for nbr in (left, right):
            pl.semaphore_signal(
                barrier_sem, inc=1,
                device_id=(nbr,), device_id_type=pl.DeviceIdType.MESH,
            )
        pl.semaphore_wait(barrier_sem, 2)

        xf_ref[pl.ds(my * m_per, m_per), :] = x_ref[...]

        for h in range(N_DEV - 1):
            src_o = lax.rem(my + N_DEV - h, N_DEV)
            in_o = lax.rem(my + N_DEV - 1 - h, N_DEV)
            send = pltpu.make_async_remote_copy(
                src_ref=xf_ref.at[pl.ds(src_o * m_per, m_per), :],
                dst_ref=xf_ref.at[pl.ds(src_o * m_per, m_per), :],
                send_sem=send_sems.at[h],
                recv_sem=recv_sems.at[h],
                device_id=(right,),
                device_id_type=pl.DeviceIdType.MESH,
            )
            send.start()
            recv = pltpu.make_async_remote_copy(
                src_ref=xf_ref.at[pl.ds(in_o * m_per, m_per), :],
                dst_ref=xf_ref.at[pl.ds(in_o * m_per, m_per), :],
                send_sem=send_sems.at[h],
                recv_sem=recv_sems.at[h],
                device_id=(left,),
                device_id_type=pl.DeviceIdType.MESH,
            )
            recv.wait_recv()
            send.wait_send()

        acc = jnp.dot(xf_ref[...], w_ref[...], preferred_element_type=jnp.int32)
        out_ref[...] = acc.astype(jnp.float32) * (sx_ref[0] * sw_ref[0])

    return pl.pallas_call(
        body,
        out_shape=jax.ShapeDtypeStruct((N_DEV * m_per, n_per), jnp.float32),
        in_specs=[
            pl.BlockSpec(memory_space=pltpu.VMEM),
            pl.BlockSpec(memory_space=pltpu.VMEM),
            pl.BlockSpec(memory_space=pltpu.SMEM),
            pl.BlockSpec(memory_space=pltpu.SMEM),
        ],
        out_specs=pl.BlockSpec(memory_space=pltpu.VMEM),
        scratch_shapes=[
            pltpu.VMEM((N_DEV * m_per, k), jnp.int8),
            pltpu.SemaphoreType.DMA((N_DEV - 1,)),
            pltpu.SemaphoreType.DMA((N_DEV - 1,)),
        ],
        compiler_params=pltpu.CompilerParams(collective_id=0),
    )(x, w_mat, scale_x, scale_w)


# device time: 141140 ns/iter; 1.7657x vs baseline; 1.7657x over previous
import jax
import jax.numpy as jnp
import numpy as np
from jax import lax
from jax.experimental import pallas as pl
from jax.experimental.pallas import tpu as pltpu

N_DEV = 32
FWD_HOPS = 16
REV_HOPS = 15

_PLANE_ORDER = [(0, 0), (1, 0), (1, 1), (0, 1), (0, 2), (1, 2), (1, 3), (0, 3)]
_LID = {}
for _z in range(4):
    for _j, (_x, _y) in enumerate(_PLANE_ORDER):
        _LID[(_x, _y, _z)] = 8 * _z + _j

_CYCLE_COORDS = []
for _y in range(4):
    _zs = range(4) if _y % 2 == 0 else range(3, -1, -1)
    _CYCLE_COORDS += [(0, _y, _z) for _z in _zs]
for _y in range(3, -1, -1):
    _zs = range(4) if _y % 2 == 1 else range(3, -1, -1)
    _CYCLE_COORDS += [(1, _y, _z) for _z in _zs]
assert len(_CYCLE_COORDS) == N_DEV
for _a, _b in zip(_CYCLE_COORDS, _CYCLE_COORDS[1:] + _CYCLE_COORDS[:1]):
    assert sum(abs(_a[_i] - _b[_i]) for _i in range(3)) == 1, (_a, _b)

_CYC = [_LID[c] for c in _CYCLE_COORDS]
_POS = {l: p for p, l in enumerate(_CYC)}

_succ = [_CYC[(_POS[l] + 1) % N_DEV] for l in range(N_DEV)]
_pred = [_CYC[(_POS[l] - 1) % N_DEV] for l in range(N_DEV)]
_fwd_send = [[_CYC[(_POS[l] - h) % N_DEV] for h in range(FWD_HOPS)]
             for l in range(N_DEV)]
_fwd_recv = [[_CYC[(_POS[l] - 1 - h) % N_DEV] for h in range(FWD_HOPS)]
             for l in range(N_DEV)]
_rev_send = [[_CYC[(_POS[l] + h) % N_DEV] for h in range(REV_HOPS)]
             for l in range(N_DEV)]
_rev_recv = [[_CYC[(_POS[l] + 1 + h) % N_DEV] for h in range(REV_HOPS)]
             for l in range(N_DEV)]

_SUCC_T = jnp.asarray(np.array(_succ, np.int32))
_PRED_T = jnp.asarray(np.array(_pred, np.int32))
_FS_T = jnp.asarray(np.array(_fwd_send, np.int32))
_FR_T = jnp.asarray(np.array(_fwd_recv, np.int32))
_RS_T = jnp.asarray(np.array(_rev_send, np.int32))
_RR_T = jnp.asarray(np.array(_rev_recv, np.int32))


def kernel(x, w_mat, scale_x, scale_w):
    m_per, k = x.shape
    _, n_per = w_mat.shape

    my = lax.axis_index("i")
    succ = jnp.take(_SUCC_T, my).reshape(1)
    pred = jnp.take(_PRED_T, my).reshape(1)
    fs = jnp.take(_FS_T, my, axis=0)
    fr = jnp.take(_FR_T, my, axis=0)
    rs = jnp.take(_RS_T, my, axis=0)
    rr = jnp.take(_RR_T, my, axis=0)

    def body(x_ref, w_ref, sx_ref, sw_ref, succ_ref, pred_ref,
             fs_ref, fr_ref, rs_ref, rr_ref, out_ref, xf_ref,
             fsend_sems, frecv_sems, rsend_sems, rrecv_sems):
        me = lax.axis_index("i")
        nxt = succ_ref[0]
        prv = pred_ref[0]

        barrier_sem = pltpu.get_barrier_semaphore()
        for nbr in (prv, nxt):
            pl.semaphore_signal(
                barrier_sem, inc=1,
                device_id=(nbr,), device_id_type=pl.DeviceIdType.MESH,
            )
        pl.semaphore_wait(barrier_sem, 2)

        xf_ref[pl.ds(me * m_per, m_per), :] = x_ref[...]

        def slot(ref, o):
            return ref.at[pl.ds(o * m_per, m_per), :]

        for h in range(FWD_HOPS):
            fsend = pltpu.make_async_remote_copy(
                src_ref=slot(xf_ref, fs_ref[h]),
                dst_ref=slot(xf_ref, fs_ref[h]),
                send_sem=fsend_sems.at[h], recv_sem=frecv_sems.at[h],
                device_id=(nxt,), device_id_type=pl.DeviceIdType.MESH,
            )
            fsend.start()
            if h < REV_HOPS:
                rsend = pltpu.make_async_remote_copy(
                    src_ref=slot(xf_ref, rs_ref[h]),
                    dst_ref=slot(xf_ref, rs_ref[h]),
                    send_sem=rsend_sems.at[h], recv_sem=rrecv_sems.at[h],
                    device_id=(prv,), device_id_type=pl.DeviceIdType.MESH,
                )
                rsend.start()
            frecv = pltpu.make_async_remote_copy(
                src_ref=slot(xf_ref, fr_ref[h]),
                dst_ref=slot(xf_ref, fr_ref[h]),
                send_sem=fsend_sems.at[h], recv_sem=frecv_sems.at[h],
                device_id=(prv,), device_id_type=pl.DeviceIdType.MESH,
            )
            frecv.wait_recv()
            fsend.wait_send()
            if h < REV_HOPS:
                rrecv = pltpu.make_async_remote_copy(
                    src_ref=slot(xf_ref, rr_ref[h]),
                    dst_ref=slot(xf_ref, rr_ref[h]),
                    send_sem=rsend_sems.at[h], recv_sem=rrecv_sems.at[h],
                    device_id=(nxt,), device_id_type=pl.DeviceIdType.MESH,
                )
                rrecv.wait_recv()
                rsend.wait_send()

        acc = jnp.dot(xf_ref[...], w_ref[...], preferred_element_type=jnp.int32)
        out_ref[...] = acc.astype(jnp.float32) * (sx_ref[0] * sw_ref[0])

    return pl.pallas_call(
        body,
        out_shape=jax.ShapeDtypeStruct((N_DEV * m_per, n_per), jnp.float32),
        in_specs=[
            pl.BlockSpec(memory_space=pltpu.VMEM),
            pl.BlockSpec(memory_space=pltpu.VMEM),
            pl.BlockSpec(memory_space=pltpu.SMEM),
            pl.BlockSpec(memory_space=pltpu.SMEM),
            pl.BlockSpec(memory_space=pltpu.SMEM),
            pl.BlockSpec(memory_space=pltpu.SMEM),
            pl.BlockSpec(memory_space=pltpu.SMEM),
            pl.BlockSpec(memory_space=pltpu.SMEM),
            pl.BlockSpec(memory_space=pltpu.SMEM),
            pl.BlockSpec(memory_space=pltpu.SMEM),
        ],
        out_specs=pl.BlockSpec(memory_space=pltpu.VMEM),
        scratch_shapes=[
            pltpu.VMEM((N_DEV * m_per, k), jnp.int8),
            pltpu.SemaphoreType.DMA((FWD_HOPS,)),
            pltpu.SemaphoreType.DMA((FWD_HOPS,)),
            pltpu.SemaphoreType.DMA((REV_HOPS,)),
            pltpu.SemaphoreType.DMA((REV_HOPS,)),
        ],
        compiler_params=pltpu.CompilerParams(collective_id=0),
    )(x, w_mat, scale_x, scale_w, succ, pred, fs, fr, rs, rr)


# device time: 115042 ns/iter; 2.1663x vs baseline; 1.2269x over previous
import jax
import jax.numpy as jnp
import numpy as np
from jax import lax
from jax.experimental import pallas as pl
from jax.experimental.pallas import tpu as pltpu

N_DEV = 32
FWD_HOPS = 16
REV_HOPS = 15
SUBS = 4

_PLANE_ORDER = [(0, 0), (1, 0), (1, 1), (0, 1), (0, 2), (1, 2), (1, 3), (0, 3)]
_LID = {}
for _z in range(4):
    for _j, (_x, _y) in enumerate(_PLANE_ORDER):
        _LID[(_x, _y, _z)] = 8 * _z + _j

_CYCLE_COORDS = []
for _y in range(4):
    _zs = range(4) if _y % 2 == 0 else range(3, -1, -1)
    _CYCLE_COORDS += [(0, _y, _z) for _z in _zs]
for _y in range(3, -1, -1):
    _zs = range(4) if _y % 2 == 1 else range(3, -1, -1)
    _CYCLE_COORDS += [(1, _y, _z) for _z in _zs]
assert len(_CYCLE_COORDS) == N_DEV
for _a, _b in zip(_CYCLE_COORDS, _CYCLE_COORDS[1:] + _CYCLE_COORDS[:1]):
    assert sum(abs(_a[_i] - _b[_i]) for _i in range(3)) == 1, (_a, _b)

_CYC = [_LID[c] for c in _CYCLE_COORDS]
_POS = {l: p for p, l in enumerate(_CYC)}

_succ = [_CYC[(_POS[l] + 1) % N_DEV] for l in range(N_DEV)]
_pred = [_CYC[(_POS[l] - 1) % N_DEV] for l in range(N_DEV)]
_fwd_send = [[_CYC[(_POS[l] - h) % N_DEV] for h in range(FWD_HOPS)]
             for l in range(N_DEV)]
_fwd_recv = [[_CYC[(_POS[l] - 1 - h) % N_DEV] for h in range(FWD_HOPS)]
             for l in range(N_DEV)]
_rev_send = [[_CYC[(_POS[l] + h) % N_DEV] for h in range(REV_HOPS)]
             for l in range(N_DEV)]
_rev_recv = [[_CYC[(_POS[l] + 1 + h) % N_DEV] for h in range(REV_HOPS)]
             for l in range(N_DEV)]

_SUCC_T = jnp.asarray(np.array(_succ, np.int32))
_PRED_T = jnp.asarray(np.array(_pred, np.int32))
_FS_T = jnp.asarray(np.array(_fwd_send, np.int32))
_FR_T = jnp.asarray(np.array(_fwd_recv, np.int32))
_RS_T = jnp.asarray(np.array(_rev_send, np.int32))
_RR_T = jnp.asarray(np.array(_rev_recv, np.int32))


def kernel(x, w_mat, scale_x, scale_w):
    m_per, k = x.shape
    _, n_per = w_mat.shape

    my = lax.axis_index("i")
    succ = jnp.take(_SUCC_T, my).reshape(1)
    pred = jnp.take(_PRED_T, my).reshape(1)
    fs = jnp.take(_FS_T, my, axis=0)
    fr = jnp.take(_FR_T, my, axis=0)
    rs = jnp.take(_RS_T, my, axis=0)
    rr = jnp.take(_RR_T, my, axis=0)

    def body(x_ref, w_ref, sx_ref, sw_ref, succ_ref, pred_ref,
             fs_ref, fr_ref, rs_ref, rr_ref, out_ref, xf_ref,
             fsend_sems, frecv_sems, rsend_sems, rrecv_sems):
        me = lax.axis_index("i")
        nxt = succ_ref[0]
        prv = pred_ref[0]

        barrier_sem = pltpu.get_barrier_semaphore()
        for nbr in (prv, nxt):
            pl.semaphore_signal(
                barrier_sem, inc=1,
                device_id=(nbr,), device_id_type=pl.DeviceIdType.MESH,
            )
        pl.semaphore_wait(barrier_sem, 2)

        xf_ref[pl.ds(me * m_per, m_per), :] = x_ref[...]

        sub = m_per // SUBS

        def subslot(o, j):
            return xf_ref.at[pl.ds(o * m_per + j * sub, sub), :]

        def fsend_d(h, j):
            return pltpu.make_async_remote_copy(
                src_ref=subslot(fs_ref[h], j), dst_ref=subslot(fs_ref[h], j),
                send_sem=fsend_sems.at[h, j], recv_sem=frecv_sems.at[h, j],
                device_id=(nxt,), device_id_type=pl.DeviceIdType.MESH,
            )

        def rsend_d(h, j):
            return pltpu.make_async_remote_copy(
                src_ref=subslot(rs_ref[h], j), dst_ref=subslot(rs_ref[h], j),
                send_sem=rsend_sems.at[h, j], recv_sem=rrecv_sems.at[h, j],
                device_id=(prv,), device_id_type=pl.DeviceIdType.MESH,
            )

        def frecv_d(h, j):
            return pltpu.make_async_remote_copy(
                src_ref=subslot(fr_ref[h], j), dst_ref=subslot(fr_ref[h], j),
                send_sem=fsend_sems.at[h, j], recv_sem=frecv_sems.at[h, j],
                device_id=(prv,), device_id_type=pl.DeviceIdType.MESH,
            )

        def rrecv_d(h, j):
            return pltpu.make_async_remote_copy(
                src_ref=subslot(rr_ref[h], j), dst_ref=subslot(rr_ref[h], j),
                send_sem=rsend_sems.at[h, j], recv_sem=rrecv_sems.at[h, j],
                device_id=(nxt,), device_id_type=pl.DeviceIdType.MESH,
            )

        for j in range(SUBS):
            fsend_d(0, j).start()
        for j in range(SUBS):
            rsend_d(0, j).start()

        for h in range(FWD_HOPS):
            for j in range(SUBS):
                frecv_d(h, j).wait_recv()
                if h + 1 < FWD_HOPS:
                    fsend_d(h + 1, j).start()
            if h < REV_HOPS:
                for j in range(SUBS):
                    rrecv_d(h, j).wait_recv()
                    if h + 1 < REV_HOPS:
                        rsend_d(h + 1, j).start()

        for h in range(FWD_HOPS):
            for j in range(SUBS):
                fsend_d(h, j).wait_send()
        for h in range(REV_HOPS):
            for j in range(SUBS):
                rsend_d(h, j).wait_send()

        acc = jnp.dot(xf_ref[...], w_ref[...], preferred_element_type=jnp.int32)
        out_ref[...] = acc.astype(jnp.float32) * (sx_ref[0] * sw_ref[0])

    return pl.pallas_call(
        body,
        out_shape=jax.ShapeDtypeStruct((N_DEV * m_per, n_per), jnp.float32),
        in_specs=[
            pl.BlockSpec(memory_space=pltpu.VMEM),
            pl.BlockSpec(memory_space=pltpu.VMEM),
            pl.BlockSpec(memory_space=pltpu.SMEM),
            pl.BlockSpec(memory_space=pltpu.SMEM),
            pl.BlockSpec(memory_space=pltpu.SMEM),
            pl.BlockSpec(memory_space=pltpu.SMEM),
            pl.BlockSpec(memory_space=pltpu.SMEM),
            pl.BlockSpec(memory_space=pltpu.SMEM),
            pl.BlockSpec(memory_space=pltpu.SMEM),
            pl.BlockSpec(memory_space=pltpu.SMEM),
        ],
        out_specs=pl.BlockSpec(memory_space=pltpu.VMEM),
        scratch_shapes=[
            pltpu.VMEM((N_DEV * m_per, k), jnp.int8),
            pltpu.SemaphoreType.DMA((FWD_HOPS, SUBS)),
            pltpu.SemaphoreType.DMA((FWD_HOPS, SUBS)),
            pltpu.SemaphoreType.DMA((REV_HOPS, SUBS)),
            pltpu.SemaphoreType.DMA((REV_HOPS, SUBS)),
        ],
        compiler_params=pltpu.CompilerParams(collective_id=0),
    )(x, w_mat, scale_x, scale_w, succ, pred, fs, fr, rs, rr)


# device time: 104745 ns/iter; 2.3792x vs baseline; 1.0983x over previous
import jax
import jax.numpy as jnp
import numpy as np
from jax import lax
from jax.experimental import pallas as pl
from jax.experimental.pallas import tpu as pltpu

N_DEV = 32
HOPS = 16
SUBS = 4

_PLANE_ORDER = [(0, 0), (1, 0), (1, 1), (0, 1), (0, 2), (1, 2), (1, 3), (0, 3)]
_LID = {}
for _z in range(4):
    for _j, (_x, _y) in enumerate(_PLANE_ORDER):
        _LID[(_x, _y, _z)] = 8 * _z + _j

_CYCLE_COORDS = []
for _y in range(4):
    _zs = range(4) if _y % 2 == 0 else range(3, -1, -1)
    _CYCLE_COORDS += [(0, _y, _z) for _z in _zs]
for _y in range(3, -1, -1):
    _zs = range(4) if _y % 2 == 1 else range(3, -1, -1)
    _CYCLE_COORDS += [(1, _y, _z) for _z in _zs]
assert len(_CYCLE_COORDS) == N_DEV
for _a, _b in zip(_CYCLE_COORDS, _CYCLE_COORDS[1:] + _CYCLE_COORDS[:1]):
    assert sum(abs(_a[_i] - _b[_i]) for _i in range(3)) == 1, (_a, _b)

_CYC = [_LID[c] for c in _CYCLE_COORDS]
_POS = {l: p for p, l in enumerate(_CYC)}

_succ = [_CYC[(_POS[l] + 1) % N_DEV] for l in range(N_DEV)]
_pred = [_CYC[(_POS[l] - 1) % N_DEV] for l in range(N_DEV)]
_fwd_send = [[_CYC[(_POS[l] - h) % N_DEV] for h in range(HOPS)]
             for l in range(N_DEV)]
_fwd_recv = [[_CYC[(_POS[l] - 1 - h) % N_DEV] for h in range(HOPS)]
             for l in range(N_DEV)]
_rev_send = [[_CYC[(_POS[l] + h) % N_DEV] for h in range(HOPS)]
             for l in range(N_DEV)]
_rev_recv = [[_CYC[(_POS[l] + 1 + h) % N_DEV] for h in range(HOPS)]
             for l in range(N_DEV)]

_SUCC_T = jnp.asarray(np.array(_succ, np.int32))
_PRED_T = jnp.asarray(np.array(_pred, np.int32))
_FS_T = jnp.asarray(np.array(_fwd_send, np.int32))
_FR_T = jnp.asarray(np.array(_fwd_recv, np.int32))
_RS_T = jnp.asarray(np.array(_rev_send, np.int32))
_RR_T = jnp.asarray(np.array(_rev_recv, np.int32))


def _f_valid(h, j):
    return 0 <= h < HOPS and (h < HOPS - 1 or j < SUBS // 2)


def _r_valid(h, j):
    return 0 <= h < HOPS and (h < HOPS - 1 or j >= SUBS // 2)


def kernel(x, w_mat, scale_x, scale_w):
    m_per, k = x.shape
    _, n_per = w_mat.shape

    my = lax.axis_index("i")
    succ = jnp.take(_SUCC_T, my).reshape(1)
    pred = jnp.take(_PRED_T, my).reshape(1)
    fs = jnp.take(_FS_T, my, axis=0)
    fr = jnp.take(_FR_T, my, axis=0)
    rs = jnp.take(_RS_T, my, axis=0)
    rr = jnp.take(_RR_T, my, axis=0)

    def body(x_ref, w_ref, sx_ref, sw_ref, succ_ref, pred_ref,
             fs_ref, fr_ref, rs_ref, rr_ref, out_ref, xf_ref,
             fsend_sems, frecv_sems, rsend_sems, rrecv_sems):
        me = lax.axis_index("i")
        nxt = succ_ref[0]
        prv = pred_ref[0]
        sub = m_per // SUBS
        scale = sx_ref[0] * sw_ref[0]

        barrier_sem = pltpu.get_barrier_semaphore()
        for nbr in (prv, nxt):
            pl.semaphore_signal(
                barrier_sem, inc=1,
                device_id=(nbr,), device_id_type=pl.DeviceIdType.MESH,
            )
        pl.semaphore_wait(barrier_sem, 2)

        def subslot(o, j):
            return xf_ref.at[pl.ds(o * m_per + j * sub, sub), :]

        def fsend_d(h, j):
            src = (x_ref.at[pl.ds(j * sub, sub), :] if h == 0
                   else subslot(fs_ref[h], j))
            return pltpu.make_async_remote_copy(
                src_ref=src, dst_ref=subslot(fs_ref[h], j),
                send_sem=fsend_sems.at[h, j], recv_sem=frecv_sems.at[h, j],
                device_id=(nxt,), device_id_type=pl.DeviceIdType.MESH,
            )

        def rsend_d(h, j):
            src = (x_ref.at[pl.ds(j * sub, sub), :] if h == 0
                   else subslot(rs_ref[h], j))
            return pltpu.make_async_remote_copy(
                src_ref=src, dst_ref=subslot(rs_ref[h], j),
                send_sem=rsend_sems.at[h, j], recv_sem=rrecv_sems.at[h, j],
                device_id=(prv,), device_id_type=pl.DeviceIdType.MESH,
            )

        def frecv_d(h, j):
            return pltpu.make_async_remote_copy(
                src_ref=subslot(fr_ref[h], j), dst_ref=subslot(fr_ref[h], j),
                send_sem=fsend_sems.at[h, j], recv_sem=frecv_sems.at[h, j],
                device_id=(prv,), device_id_type=pl.DeviceIdType.MESH,
            )

        def rrecv_d(h, j):
            return pltpu.make_async_remote_copy(
                src_ref=subslot(rr_ref[h], j), dst_ref=subslot(rr_ref[h], j),
                send_sem=rsend_sems.at[h, j], recv_sem=rrecv_sems.at[h, j],
                device_id=(nxt,), device_id_type=pl.DeviceIdType.MESH,
            )

        def chunk_gemm(o, chunk):
            acc = jnp.dot(chunk, w_ref[...], preferred_element_type=jnp.int32)
            out_ref[pl.ds(o * m_per, m_per), :] = acc.astype(jnp.float32) * scale

        for j in range(SUBS):
            fsend_d(0, j).start()
        for j in range(SUBS):
            rsend_d(0, j).start()

        chunk_gemm(me, x_ref[...])

        for h in range(HOPS):
            for j in range(SUBS):
                if _f_valid(h, j):
                    frecv_d(h, j).wait_recv()
                    if _f_valid(h + 1, j):
                        fsend_d(h + 1, j).start()
                if _r_valid(h, j):
                    rrecv_d(h, j).wait_recv()
                    if _r_valid(h + 1, j):
                        rsend_d(h + 1, j).start()
            if h < HOPS - 1:
                chunk_gemm(fr_ref[h], xf_ref[pl.ds(fr_ref[h] * m_per, m_per), :])
                chunk_gemm(rr_ref[h], xf_ref[pl.ds(rr_ref[h] * m_per, m_per), :])
            else:
                chunk_gemm(fr_ref[h], xf_ref[pl.ds(fr_ref[h] * m_per, m_per), :])

        for h in range(HOPS):
            for j in range(SUBS):
                if _f_valid(h, j):
                    fsend_d(h, j).wait_send()
                if _r_valid(h, j):
                    rsend_d(h, j).wait_send()

    return pl.pallas_call(
        body,
        out_shape=jax.ShapeDtypeStruct((N_DEV * m_per, n_per), jnp.float32),
        in_specs=[
            pl.BlockSpec(memory_space=pltpu.VMEM),
            pl.BlockSpec(memory_space=pltpu.VMEM),
            pl.BlockSpec(memory_space=pltpu.SMEM),
            pl.BlockSpec(memory_space=pltpu.SMEM),
            pl.BlockSpec(memory_space=pltpu.SMEM),
            pl.BlockSpec(memory_space=pltpu.SMEM),
            pl.BlockSpec(memory_space=pltpu.SMEM),
            pl.BlockSpec(memory_space=pltpu.SMEM),
            pl.BlockSpec(memory_space=pltpu.SMEM),
            pl.BlockSpec(memory_space=pltpu.SMEM),
        ],
        out_specs=pl.BlockSpec(memory_space=pltpu.VMEM),
        scratch_shapes=[
            pltpu.VMEM((N_DEV * m_per, k), jnp.int8),
            pltpu.SemaphoreType.DMA((HOPS, SUBS)),
            pltpu.SemaphoreType.DMA((HOPS, SUBS)),
            pltpu.SemaphoreType.DMA((HOPS, SUBS)),
            pltpu.SemaphoreType.DMA((HOPS, SUBS)),
        ],
        compiler_params=pltpu.CompilerParams(collective_id=0),
    )(x, w_mat, scale_x, scale_w, succ, pred, fs, fr, rs, rr)
